# baseline (speedup 1.0000x reference)
import functools

import jax
import jax.numpy as jnp
from jax import lax
from jax.experimental import pallas as pl
from jax.experimental.pallas import tpu as pltpu
from jax.experimental.pallas import tpu_sc as plsc

B = 4096
D = 32
G = 16
NC, NS = 2, 16
NW = NC * NS
BPW = B // NW
ROWS_BLK = 256


def _sc_gather(user_emb, item_emb, ub16, ib16, uidx, iidx, uidx16, iidx16):
    f32 = jnp.float32
    mesh = plsc.VectorSubcoreMesh(core_axis_name="c", subcore_axis_name="s")

    @functools.partial(
        pl.kernel,
        compiler_params=pltpu.CompilerParams(use_tc_tiling_on_sc=False),
        out_type=[
            jax.ShapeDtypeStruct((B, D), f32),
            jax.ShapeDtypeStruct((B, D), f32),
            jax.ShapeDtypeStruct((B, G), f32),
            jax.ShapeDtypeStruct((B, G), f32),
        ],
        mesh=mesh,
        scratch_types=[
            pltpu.VMEM((BPW,), jnp.int32),
            pltpu.VMEM((BPW,), jnp.int32),
            pltpu.VMEM((BPW,), jnp.int32),
            pltpu.VMEM((BPW,), jnp.int32),
            pltpu.VMEM((BPW, D), f32),
            pltpu.VMEM((BPW, D), f32),
            pltpu.VMEM((BPW, G), f32),
            pltpu.VMEM((BPW, G), f32),
            pltpu.SemaphoreType.DMA,
            pltpu.SemaphoreType.DMA,
            pltpu.SemaphoreType.DMA,
            pltpu.SemaphoreType.DMA,
        ],
    )
    def k(ue_hbm, ie_hbm, ub_hbm, ib_hbm, ui_hbm, ii_hbm, ui16_hbm, ii16_hbm,
          ue_out, ie_out, ubr_out, ibr_out,
          uidx_v, iidx_v, uidx16_v, iidx16_v, urows_v, irows_v, ub_v, ib_v,
          sem_u, sem_i, sem_ub, sem_ib):
        wid = lax.axis_index("s") * NC + lax.axis_index("c")
        base = wid * BPW
        pltpu.sync_copy(ui_hbm.at[pl.ds(base, BPW)], uidx_v)
        pltpu.sync_copy(ii_hbm.at[pl.ds(base, BPW)], iidx_v)
        pltpu.sync_copy(ui16_hbm.at[pl.ds(base, BPW)], uidx16_v)
        pltpu.sync_copy(ii16_hbm.at[pl.ds(base, BPW)], iidx16_v)
        cu = pltpu.async_copy(ue_hbm.at[uidx_v], urows_v, sem_u)
        ci = pltpu.async_copy(ie_hbm.at[iidx_v], irows_v, sem_i)
        cub = pltpu.async_copy(ub_hbm.at[uidx16_v], ub_v, sem_ub)
        cib = pltpu.async_copy(ib_hbm.at[iidx16_v], ib_v, sem_ib)
        cu.wait()
        ci.wait()
        cub.wait()
        cib.wait()
        pltpu.sync_copy(urows_v, ue_out.at[pl.ds(base, BPW)])
        pltpu.sync_copy(irows_v, ie_out.at[pl.ds(base, BPW)])
        pltpu.sync_copy(ub_v, ubr_out.at[pl.ds(base, BPW)])
        pltpu.sync_copy(ib_v, ibr_out.at[pl.ds(base, BPW)])

    return k(user_emb, item_emb, ub16, ib16, uidx, iidx, uidx16, iidx16)


def _tc_body(ue_ref, ie_ref, ubr_ref, ibr_ref, umod_ref, imod_ref,
             out_ref, sig_ref):
    @pl.when(pl.program_id(0) == 0)
    def _():
        sig_ref[...] = jnp.sum(ue_ref[...] * ie_ref[...], axis=1)[None, :]

    lane = lax.broadcasted_iota(jnp.int32, (ROWS_BLK, G), 1)
    ub = jnp.where(lane == umod_ref[...], ubr_ref[...], 0.0).sum(
        axis=1, keepdims=True)
    ib = jnp.where(lane == imod_ref[...], ibr_ref[...], 0.0).sum(
        axis=1, keepdims=True)
    out_ref[...] = sig_ref[...] + (ub + ib)


def kernel(user_ids, item_ids, user_emb, item_emb, user_bias, item_bias):
    uidx = user_ids.astype(jnp.int32)
    iidx = item_ids.astype(jnp.int32)
    uidx16 = uidx // G
    iidx16 = iidx // G
    umod = (uidx % G).reshape(B, 1)
    imod = (iidx % G).reshape(B, 1)
    ub16 = user_bias.reshape(-1, G)
    ib16 = item_bias.reshape(-1, G)
    ue, ie, ubr, ibr = _sc_gather(
        user_emb, item_emb, ub16, ib16, uidx, iidx, uidx16, iidx16)
    out = pl.pallas_call(
        _tc_body,
        grid=(B // ROWS_BLK,),
        in_specs=[
            pl.BlockSpec((B, D), lambda i: (0, 0)),
            pl.BlockSpec((B, D), lambda i: (0, 0)),
            pl.BlockSpec((ROWS_BLK, G), lambda i: (i, 0)),
            pl.BlockSpec((ROWS_BLK, G), lambda i: (i, 0)),
            pl.BlockSpec((ROWS_BLK, 1), lambda i: (i, 0)),
            pl.BlockSpec((ROWS_BLK, 1), lambda i: (i, 0)),
        ],
        out_specs=pl.BlockSpec((ROWS_BLK, B), lambda i: (i, 0)),
        out_shape=jax.ShapeDtypeStruct((B, B), jnp.float32),
        scratch_shapes=[pltpu.VMEM((1, B), jnp.float32)],
    )(ue, ie, ubr, ibr, umod, imod)
    return out
